# baseline (speedup 1.0000x reference)
import dataclasses
import functools

import jax
import jax.numpy as jnp
from jax import lax
from jax.experimental import pallas as pl
from jax.experimental.pallas import tpu as pltpu
from jax.experimental.pallas import tpu_sc as plsc

EMBED = 64
LANES = 16
NCORE = 2
NSUB = 16
NWORK = NCORE * NSUB
BATCH = 16384
BPW = BATCH // NWORK
GCHUNK = 128
NCHUNK = BPW // GCHUNK
GROUP = LANES
NGROUP = BPW // GROUP
KCH = EMBED // LANES

_cp = pltpu.CompilerParams(use_tc_tiling_on_sc=False)
if "needs_layout_passes" in pltpu.CompilerParams.__dataclass_fields__:
    _cp = dataclasses.replace(_cp, needs_layout_passes=False)


@functools.partial(
    pl.kernel,
    compiler_params=_cp,
    out_type=jax.ShapeDtypeStruct((BATCH,), jnp.float32),
    mesh=plsc.VectorSubcoreMesh(core_axis_name="c", subcore_axis_name="s"),
    scratch_types=[
        pltpu.VMEM((BPW,), jnp.int32),
        pltpu.VMEM((BPW,), jnp.int32),
        pltpu.VMEM((BPW, EMBED), jnp.float32),
        pltpu.VMEM((BPW, EMBED), jnp.float32),
        pltpu.VMEM((GROUP, LANES), jnp.float32),
        pltpu.VMEM((BPW,), jnp.float32),
        pltpu.SemaphoreType.DMA,
    ],
)
def _sgns_sc(c_hbm, w_hbm, ctab_hbm, wtab_hbm, out_hbm,
             cidx, widx, crows, wrows, accbuf, outv, sem):
    wid = lax.axis_index("s") * NCORE + lax.axis_index("c")
    base = wid * BPW

    pltpu.sync_copy(c_hbm.at[pl.ds(base, BPW)], cidx)
    pltpu.sync_copy(w_hbm.at[pl.ds(base, BPW)], widx)

    copies = []
    for k in range(NCHUNK):
        sl = pl.ds(k * GCHUNK, GCHUNK)
        copies.append(pltpu.async_copy(ctab_hbm.at[cidx.at[sl]], crows.at[sl], sem))
        copies.append(pltpu.async_copy(wtab_hbm.at[widx.at[sl]], wrows.at[sl], sem))
    for cp in copies:
        cp.wait()

    row_iota = lax.iota(jnp.int32, LANES)
    one = jnp.full((LANES,), 1.0, jnp.float32)

    @pl.loop(0, NGROUP)
    def _(g):
        g0 = g * GROUP
        for r in range(GROUP):
            row = g0 + r
            acc = None
            for k in range(KCH):
                cv = crows[row, pl.ds(k * LANES, LANES)]
                wv = wrows[row, pl.ds(k * LANES, LANES)]
                p = cv * wv
                acc = p if acc is None else acc + p
            accbuf[r, :] = acc
        tot = None
        for j in range(LANES):
            col = plsc.load_gather(
                accbuf, [row_iota, jnp.full((LANES,), j, jnp.int32)])
            tot = col if tot is None else tot + col
        outv[pl.ds(g0, GROUP)] = one / (one + jnp.exp(-tot))

    pltpu.sync_copy(outv, out_hbm.at[pl.ds(base, BPW)])


def kernel(c, w, c_table, w_table):
    return _sgns_sc(c, w, c_table, w_table)
